# Initial kernel scaffold; baseline (speedup 1.0000x reference)
import jax
import jax.numpy as jnp
from jax.experimental import pallas as pl


def kernel(x, edge_index, W1l, b1, W1r, W2l, b2, W2r):
    raise NotImplementedError("write your pallas kernel here")


# speedup vs baseline: 9.3829x; 9.3829x over previous
import functools

import jax
import jax.numpy as jnp
from jax import lax
from jax.experimental import pallas as pl
from jax.experimental.pallas import tpu as pltpu
from jax.experimental.pallas import tpu_sc as plsc

NN = 10000
NP = 10112
EE = 320000
CH = 128
NW = 32
RPT = 80
EP = NW * RPT * CH
NROW = EP // CH
RS = NP // 16
OP = 384
OO = 300


def _seg_body(with_deg, vals, srcs, dsts, zeros_h, ones_h, *rest):
    if with_deg:
        out_acc, out_deg, src_v, dst_v, rows_v, ones_v, acc, accd, sem = rest
    else:
        out_acc, src_v, dst_v, rows_v, ones_v, acc, accd, sem = rest
        out_deg = None
    cid = lax.axis_index("c")
    sid = lax.axis_index("s")
    wid = sid * 2 + cid
    pltpu.sync_copy(zeros_h.at[pl.ds(sid * RS, RS)], acc.at[pl.ds(sid * RS, RS)])
    if with_deg:
        pltpu.sync_copy(zeros_h.at[pl.ds(sid * RS, RS)], accd.at[pl.ds(sid * RS, RS)])
        pltpu.sync_copy(ones_h, ones_v)
    pltpu.sync_copy(srcs.at[pl.ds(wid * RPT, RPT)], src_v)
    pltpu.sync_copy(dsts.at[pl.ds(wid * RPT, RPT)], dst_v)
    plsc.subcore_barrier()

    def step(j, carry):
        pltpu.async_copy(vals.at[src_v.at[j]], rows_v, sem).wait()
        pltpu.sync_copy(rows_v, acc.at[dst_v.at[j]], add=True)
        if with_deg:
            pltpu.sync_copy(ones_v, accd.at[dst_v.at[j]], add=True)
        return carry

    lax.fori_loop(0, RPT, step, 0)
    plsc.subcore_barrier()
    row0 = cid * NP + sid * RS
    pltpu.sync_copy(acc.at[pl.ds(sid * RS, RS)], out_acc.at[pl.ds(row0, RS)])
    if with_deg:
        pltpu.sync_copy(accd.at[pl.ds(sid * RS, RS)], out_deg.at[pl.ds(row0, RS)])


@functools.cache
def _make_seg(with_deg):
    mesh = plsc.VectorSubcoreMesh(
        core_axis_name="c", subcore_axis_name="s", num_cores=2, num_subcores=16
    )
    outs = [jax.ShapeDtypeStruct((2 * NP, 16), jnp.float32)]
    if with_deg:
        outs.append(jax.ShapeDtypeStruct((2 * NP, 16), jnp.float32))
    return pl.kernel(
        functools.partial(_seg_body, with_deg),
        out_type=tuple(outs) if with_deg else outs[0],
        mesh=mesh,
        scratch_types=[
            pltpu.VMEM((RPT, CH), jnp.int32),
            pltpu.VMEM((RPT, CH), jnp.int32),
            pltpu.VMEM((CH, 16), jnp.float32),
            pltpu.VMEM((CH, 16), jnp.float32),
            pltpu.VMEM_SHARED((NP, 16), jnp.float32),
            pltpu.VMEM_SHARED((NP, 16), jnp.float32),
            pltpu.SemaphoreType.DMA,
        ],
        compiler_params=pltpu.CompilerParams(use_tc_tiling_on_sc=False),
    )


def _proj_body(x_ref, wl_ref, wr_ref, xl_ref, xr_ref):
    x = x_ref[...]
    xl_ref[...] = jnp.dot(x, wl_ref[...], preferred_element_type=jnp.float32)
    xr_ref[...] = jnp.dot(x, wr_ref[...], preferred_element_type=jnp.float32)


_proj = pl.pallas_call(
    _proj_body,
    out_shape=(
        jax.ShapeDtypeStruct((NP, 16), jnp.float32),
        jax.ShapeDtypeStruct((NP, 16), jnp.float32),
    ),
)


def _h_body(a_ref, d_ref, xr_ref, b1_ref, h_ref, dinv_ref):
    deg = d_ref[:NP] + d_ref[NP:]
    dinv = 1.0 / jnp.maximum(deg, 1.0)
    agg = a_ref[:NP] + a_ref[NP:]
    h_ref[...] = jnp.maximum(agg * dinv + b1_ref[...] + xr_ref[...], 0.0)
    dinv_ref[...] = dinv


_hcomb = pl.pallas_call(
    _h_body,
    out_shape=(
        jax.ShapeDtypeStruct((NP, 16), jnp.float32),
        jax.ShapeDtypeStruct((NP, 16), jnp.float32),
    ),
)

_BN = 2528


def _out_body(a_ref, dinv_ref, h_ref, w2l_ref, w2r_ref, b2_ref, o_ref):
    m2 = (a_ref[0] + a_ref[1]) * dinv_ref[...]
    z = (jnp.dot(m2, w2l_ref[...], preferred_element_type=jnp.float32)
         + jnp.dot(h_ref[...], w2r_ref[...], preferred_element_type=jnp.float32)
         + b2_ref[...])
    m = jnp.max(z, axis=1, keepdims=True)
    lse = jnp.log(jnp.sum(jnp.exp(z - m), axis=1, keepdims=True)) + m
    o_ref[...] = z - lse


_outk = pl.pallas_call(
    _out_body,
    grid=(NP // _BN,),
    in_specs=[
        pl.BlockSpec((2, _BN, 16), lambda i: (0, i, 0)),
        pl.BlockSpec((_BN, 16), lambda i: (i, 0)),
        pl.BlockSpec((_BN, 16), lambda i: (i, 0)),
        pl.BlockSpec((16, OP), lambda i: (0, 0)),
        pl.BlockSpec((16, OP), lambda i: (0, 0)),
        pl.BlockSpec((1, OP), lambda i: (0, 0)),
    ],
    out_specs=pl.BlockSpec((_BN, OP), lambda i: (i, 0)),
    out_shape=jax.ShapeDtypeStruct((NP, OP), jnp.float32),
)


def kernel(x, edge_index, W1l, b1, W1r, W2l, b2, W2r):
    src = edge_index[0].astype(jnp.int32)
    dst = edge_index[1].astype(jnp.int32)
    pad = jnp.full((EP - EE,), NN, jnp.int32)
    srcs = jnp.concatenate([src, pad]).reshape(NROW, CH)
    dsts = jnp.concatenate([dst, pad]).reshape(NROW, CH)
    xp = jnp.zeros((NP, 128), jnp.float32).at[:NN].set(x)
    zeros_h = jnp.zeros((NP, 16), jnp.float32)
    ones_h = jnp.ones((CH, 16), jnp.float32)

    xl, xr = _proj(xp, W1l.T, W1r.T)
    agg1p, degp = _make_seg(True)(xl, srcs, dsts, zeros_h, ones_h)
    h, dinv = _hcomb(agg1p, degp, xr, b1.reshape(1, 16))
    agg2p = _make_seg(False)(h, srcs, dsts, zeros_h, ones_h)

    w2l_t = jnp.zeros((16, OP), jnp.float32).at[:, :OO].set(W2l.T)
    w2r_t = jnp.zeros((16, OP), jnp.float32).at[:, :OO].set(W2r.T)
    b2p = jnp.full((1, OP), -1e30, jnp.float32).at[0, :OO].set(b2)
    out = _outk(agg2p.reshape(2, NP, 16), dinv, h, w2l_t, w2r_t, b2p)
    return out[:NN, :OO]


# speedup vs baseline: 17.1762x; 1.8306x over previous
import functools

import jax
import jax.numpy as jnp
from jax import lax
from jax.experimental import pallas as pl
from jax.experimental.pallas import tpu as pltpu
from jax.experimental.pallas import tpu_sc as plsc

NN = 10000
NP = 10112
EE = 320000
CH = 128
NROW = EE // CH
NW = 32
BASE = 78
K = 6
NG = BASE // K
RS = NP // 16
OP = 384
OO = 300


def _seg_body(with_deg, vals, edges, zeros_h, ones_h, *rest):
    if with_deg:
        out_acc, out_deg, src_v, dst_v, rows_v, ones_v, acc, accd, sem_g, sem_sv, sem_sd = rest
    else:
        out_acc, src_v, dst_v, rows_v, ones_v, acc, accd, sem_g, sem_sv, sem_sd = rest
        out_deg = None
    cid = lax.axis_index("c")
    sid = lax.axis_index("s")
    wid = sid * 2 + cid
    pltpu.sync_copy(zeros_h.at[pl.ds(sid * RS, RS)], acc.at[pl.ds(sid * RS, RS)])
    if with_deg:
        pltpu.sync_copy(zeros_h.at[pl.ds(sid * RS, RS)], accd.at[pl.ds(sid * RS, RS)])
        pltpu.sync_copy(ones_h, ones_v)
    pltpu.sync_copy(edges.at[0, pl.ds(wid * BASE, BASE)], src_v.at[pl.ds(0, BASE)])
    pltpu.sync_copy(edges.at[1, pl.ds(wid * BASE, BASE)], dst_v.at[pl.ds(0, BASE)])

    @pl.when(wid < NROW - NW * BASE)
    def _():
        pltpu.sync_copy(edges.at[0, pl.ds(NW * BASE + wid, 1)], src_v.at[pl.ds(BASE, 1)])
        pltpu.sync_copy(edges.at[1, pl.ds(NW * BASE + wid, 1)], dst_v.at[pl.ds(BASE, 1)])

    plsc.subcore_barrier()

    def gather(row, slot):
        pltpu.async_copy(vals.at[src_v.at[row]], rows_v.at[pl.ds(slot * CH, CH)], sem_g)

    def drain_gather():
        pltpu.make_async_copy(
            vals.at[src_v.at[0]], rows_v.at[pl.ds(0, CH)], sem_g).wait()

    def scatter(row, slot):
        pltpu.async_copy(rows_v.at[pl.ds(slot * CH, CH)], acc.at[dst_v.at[row]],
                         sem_sv, add=True)
        if with_deg:
            pltpu.async_copy(ones_v, accd.at[dst_v.at[row]], sem_sd, add=True)

    def drain_scatter():
        pltpu.make_async_copy(
            rows_v.at[pl.ds(0, CH)], acc.at[dst_v.at[0]], sem_sv).wait()
        if with_deg:
            pltpu.make_async_copy(ones_v, accd.at[dst_v.at[0]], sem_sd).wait()

    def group(g, carry):
        s = lax.rem(g, 2)

        @pl.when(g >= 2)
        def _():
            for _k in range(K):
                drain_scatter()

        for k in range(K):
            gather(g * K + k, s * K + k)
        for _k in range(K):
            drain_gather()
        for k in range(K):
            scatter(g * K + k, s * K + k)
        return carry

    lax.fori_loop(0, NG, group, 0)
    for _k in range(2 * K):
        drain_scatter()

    @pl.when(wid < NROW - NW * BASE)
    def _():
        pltpu.async_copy(vals.at[src_v.at[BASE]], rows_v.at[pl.ds(0, CH)], sem_g).wait()
        pltpu.sync_copy(rows_v.at[pl.ds(0, CH)], acc.at[dst_v.at[BASE]], add=True)
        if with_deg:
            pltpu.sync_copy(ones_v, accd.at[dst_v.at[BASE]], add=True)

    plsc.subcore_barrier()
    pltpu.sync_copy(acc.at[pl.ds(sid * RS, RS)], out_acc.at[cid, pl.ds(sid * RS, RS)])
    if with_deg:
        pltpu.sync_copy(accd.at[pl.ds(sid * RS, RS)], out_deg.at[cid, pl.ds(sid * RS, RS)])


@functools.cache
def _make_seg(with_deg):
    mesh = plsc.VectorSubcoreMesh(
        core_axis_name="c", subcore_axis_name="s", num_cores=2, num_subcores=16
    )
    outs = [jax.ShapeDtypeStruct((2, NP, 16), jnp.float32)]
    if with_deg:
        outs.append(jax.ShapeDtypeStruct((2, NP, 16), jnp.float32))
    return pl.kernel(
        functools.partial(_seg_body, with_deg),
        out_type=tuple(outs) if with_deg else outs[0],
        mesh=mesh,
        scratch_types=[
            pltpu.VMEM((BASE + 1, CH), jnp.int32),
            pltpu.VMEM((BASE + 1, CH), jnp.int32),
            pltpu.VMEM((2 * K * CH, 16), jnp.float32),
            pltpu.VMEM((CH, 16), jnp.float32),
            pltpu.VMEM_SHARED((NP, 16), jnp.float32),
            pltpu.VMEM_SHARED((NP, 16), jnp.float32),
            pltpu.SemaphoreType.DMA,
            pltpu.SemaphoreType.DMA,
            pltpu.SemaphoreType.DMA,
        ],
        compiler_params=pltpu.CompilerParams(use_tc_tiling_on_sc=False),
    )


def _proj_body(x_ref, wl_ref, wr_ref, xl_ref, xr_ref):
    x = x_ref[...]
    xl_ref[...] = jnp.dot(x, wl_ref[...], preferred_element_type=jnp.float32)
    xr_ref[...] = jnp.dot(x, wr_ref[...], preferred_element_type=jnp.float32)


_proj = pl.pallas_call(
    _proj_body,
    out_shape=(
        jax.ShapeDtypeStruct((NN, 16), jnp.float32),
        jax.ShapeDtypeStruct((NN, 16), jnp.float32),
    ),
)


def _h_body(a_ref, d_ref, xr_ref, b1_ref, h_ref, dinv_ref):
    deg = d_ref[0, :NN] + d_ref[1, :NN]
    dinv = 1.0 / jnp.maximum(deg, 1.0)
    agg = a_ref[0, :NN] + a_ref[1, :NN]
    h_ref[...] = jnp.maximum(agg * dinv + b1_ref[...] + xr_ref[...], 0.0)
    dinv_ref[...] = dinv


_hcomb = pl.pallas_call(
    _h_body,
    out_shape=(
        jax.ShapeDtypeStruct((NN, 16), jnp.float32),
        jax.ShapeDtypeStruct((NN, 16), jnp.float32),
    ),
)

_BN = 2000


def _out_body(a_ref, dinv_ref, h_ref, w2l_ref, w2r_ref, b2_ref, o_ref):
    m2 = (a_ref[0] + a_ref[1]) * dinv_ref[...]
    z = (jnp.dot(m2, w2l_ref[...], preferred_element_type=jnp.float32)
         + jnp.dot(h_ref[...], w2r_ref[...], preferred_element_type=jnp.float32)
         + b2_ref[...])
    m = jnp.max(z, axis=1, keepdims=True)
    lse = jnp.log(jnp.sum(jnp.exp(z - m), axis=1, keepdims=True)) + m
    o_ref[...] = z - lse


_outk = pl.pallas_call(
    _out_body,
    grid=(NN // _BN,),
    in_specs=[
        pl.BlockSpec((2, _BN, 16), lambda i: (0, i, 0)),
        pl.BlockSpec((_BN, 16), lambda i: (i, 0)),
        pl.BlockSpec((_BN, 16), lambda i: (i, 0)),
        pl.BlockSpec((16, OP), lambda i: (0, 0)),
        pl.BlockSpec((16, OP), lambda i: (0, 0)),
        pl.BlockSpec((1, OP), lambda i: (0, 0)),
    ],
    out_specs=pl.BlockSpec((_BN, OP), lambda i: (i, 0)),
    out_shape=jax.ShapeDtypeStruct((NN, OP), jnp.float32),
)


def kernel(x, edge_index, W1l, b1, W1r, W2l, b2, W2r):
    edges = edge_index.astype(jnp.int32).reshape(2, NROW, CH)
    zeros_h = jnp.zeros((NP, 16), jnp.float32)
    ones_h = jnp.ones((CH, 16), jnp.float32)

    xl, xr = _proj(x, W1l.T, W1r.T)
    agg1p, degp = _make_seg(True)(xl, edges, zeros_h, ones_h)
    h, dinv = _hcomb(agg1p, degp, xr, b1.reshape(1, 16))
    agg2p = _make_seg(False)(h, edges, zeros_h, ones_h)

    w2l_t = jnp.zeros((16, OP), jnp.float32).at[:, :OO].set(W2l.T)
    w2r_t = jnp.zeros((16, OP), jnp.float32).at[:, :OO].set(W2r.T)
    b2p = jnp.full((1, OP), -1e30, jnp.float32).at[0, :OO].set(b2)
    out = _outk(agg2p, dinv, h, w2l_t, w2r_t, b2p)
    return out[:, :OO]


# speedup vs baseline: 20.1421x; 1.1727x over previous
import functools

import jax
import jax.numpy as jnp
from jax import lax
from jax.experimental import pallas as pl
from jax.experimental.pallas import tpu as pltpu
from jax.experimental.pallas import tpu_sc as plsc

NN = 10000
NPK = 1250
NP = 10112
NPP = NP // 8
EE = 320000
CH = 128
NROW = EE // CH
NW = 32
BASE = 78
K = 6
NG = BASE // K
RS = NP // 16
OP = 384
OO = 300


def _seg_body(with_deg, vals, edges, zeros_h, ones_h, *rest):
    if with_deg:
        out_acc, out_deg, src_v, dst_v, rows_v, ones_v, acc, accd, sem_g, sem_sv, sem_sd = rest
    else:
        out_acc, src_v, dst_v, rows_v, ones_v, acc, accd, sem_g, sem_sv, sem_sd = rest
        out_deg = None
    cid = lax.axis_index("c")
    sid = lax.axis_index("s")
    wid = sid * 2 + cid
    pltpu.sync_copy(zeros_h.at[pl.ds(sid * RS, RS)], acc.at[pl.ds(sid * RS, RS)])
    if with_deg:
        pltpu.sync_copy(zeros_h.at[pl.ds(sid * RS, RS)], accd.at[pl.ds(sid * RS, RS)])
        pltpu.sync_copy(ones_h, ones_v)
    pltpu.sync_copy(edges.at[pl.ds(wid * BASE, BASE), 0], src_v.at[pl.ds(0, BASE)])
    pltpu.sync_copy(edges.at[pl.ds(wid * BASE, BASE), 1], dst_v.at[pl.ds(0, BASE)])

    @pl.when(wid < NROW - NW * BASE)
    def _():
        pltpu.sync_copy(edges.at[pl.ds(NW * BASE + wid, 1), 0], src_v.at[pl.ds(BASE, 1)])
        pltpu.sync_copy(edges.at[pl.ds(NW * BASE + wid, 1), 1], dst_v.at[pl.ds(BASE, 1)])

    plsc.subcore_barrier()

    def gather(row, slot):
        pltpu.async_copy(vals.at[src_v.at[row]], rows_v.at[pl.ds(slot * CH, CH)], sem_g)

    def drain_gather():
        pltpu.make_async_copy(
            vals.at[src_v.at[0]], rows_v.at[pl.ds(0, CH)], sem_g).wait()

    def scatter(row, slot):
        pltpu.async_copy(rows_v.at[pl.ds(slot * CH, CH)], acc.at[dst_v.at[row]],
                         sem_sv, add=True)
        if with_deg:
            pltpu.async_copy(ones_v, accd.at[dst_v.at[row]], sem_sd, add=True)

    def drain_scatter():
        pltpu.make_async_copy(
            rows_v.at[pl.ds(0, CH)], acc.at[dst_v.at[0]], sem_sv).wait()
        if with_deg:
            pltpu.make_async_copy(ones_v, accd.at[dst_v.at[0]], sem_sd).wait()

    def group(g, carry):
        s = lax.rem(g, 2)

        @pl.when(g >= 2)
        def _():
            for _k in range(K):
                drain_scatter()

        for k in range(K):
            gather(g * K + k, s * K + k)
        for _k in range(K):
            drain_gather()
        for k in range(K):
            scatter(g * K + k, s * K + k)
        return carry

    lax.fori_loop(0, NG, group, 0)
    for _k in range(2 * K):
        drain_scatter()

    @pl.when(wid < NROW - NW * BASE)
    def _():
        pltpu.async_copy(vals.at[src_v.at[BASE]], rows_v.at[pl.ds(0, CH)], sem_g).wait()
        pltpu.sync_copy(rows_v.at[pl.ds(0, CH)], acc.at[dst_v.at[BASE]], add=True)
        if with_deg:
            pltpu.sync_copy(ones_v, accd.at[dst_v.at[BASE]], add=True)

    plsc.subcore_barrier()
    pltpu.sync_copy(acc.at[pl.ds(sid * RS, RS)], out_acc.at[cid, pl.ds(sid * RS, RS)])
    if with_deg:
        pltpu.sync_copy(accd.at[pl.ds(sid * RS, RS)], out_deg.at[cid, pl.ds(sid * RS, RS)])


@functools.cache
def _make_seg(with_deg):
    mesh = plsc.VectorSubcoreMesh(
        core_axis_name="c", subcore_axis_name="s", num_cores=2, num_subcores=16
    )
    outs = [jax.ShapeDtypeStruct((2, NP, 16), jnp.float32)]
    if with_deg:
        outs.append(jax.ShapeDtypeStruct((2, NP, 16), jnp.float32))
    return pl.kernel(
        functools.partial(_seg_body, with_deg),
        out_type=tuple(outs) if with_deg else outs[0],
        mesh=mesh,
        scratch_types=[
            pltpu.VMEM((BASE + 1, CH), jnp.int32),
            pltpu.VMEM((BASE + 1, CH), jnp.int32),
            pltpu.VMEM((2 * K * CH, 16), jnp.float32),
            pltpu.VMEM((CH, 16), jnp.float32),
            pltpu.VMEM_SHARED((NP, 16), jnp.float32),
            pltpu.VMEM_SHARED((NP, 16), jnp.float32),
            pltpu.SemaphoreType.DMA,
            pltpu.SemaphoreType.DMA,
            pltpu.SemaphoreType.DMA,
        ],
        compiler_params=pltpu.CompilerParams(use_tc_tiling_on_sc=False),
    )


def _proj_body(x_ref, wl_ref, wr_ref, xl_ref, xr_ref):
    xv = x_ref[...]
    xls, xrs = [], []
    for a in range(8):
        xa = xv[:, a, :]
        xls.append(jnp.dot(xa, wl_ref[...], preferred_element_type=jnp.float32))
        xrs.append(jnp.dot(xa, wr_ref[...], preferred_element_type=jnp.float32))
    xl_ref[...] = jnp.concatenate(xls, axis=1)
    xr_ref[...] = jnp.concatenate(xrs, axis=1)


_proj = pl.pallas_call(
    _proj_body,
    out_shape=(
        jax.ShapeDtypeStruct((NPK, 128), jnp.float32),
        jax.ShapeDtypeStruct((NPK, 128), jnp.float32),
    ),
)


def _h_body(a_ref, d_ref, xr_ref, b1_ref, hp_ref, dinv_ref):
    deg = d_ref[0, :NPK] + d_ref[1, :NPK]
    dinv = 1.0 / jnp.maximum(deg, 1.0)
    agg = a_ref[0, :NPK] + a_ref[1, :NPK]
    hp_ref[...] = jnp.maximum(agg * dinv + b1_ref[...] + xr_ref[...], 0.0)
    dinv_ref[...] = dinv


_hcomb = pl.pallas_call(
    _h_body,
    out_shape=(
        jax.ShapeDtypeStruct((NPK, 128), jnp.float32),
        jax.ShapeDtypeStruct((NPK, 128), jnp.float32),
    ),
)


def _out_body(a_ref, dinv_ref, h_ref, w2l_ref, w2r_ref, b2_ref, o_ref):
    m2p = (a_ref[0, :NPK] + a_ref[1, :NPK]) * dinv_ref[...]
    hpv = h_ref[...]
    cols = []
    for a in range(8):
        m2a = m2p[:, 16 * a:16 * (a + 1)]
        ha = hpv[:, 16 * a:16 * (a + 1)]
        z = (jnp.dot(m2a, w2l_ref[...], preferred_element_type=jnp.float32)
             + jnp.dot(ha, w2r_ref[...], preferred_element_type=jnp.float32)
             + b2_ref[...])
        m = jnp.max(z, axis=1, keepdims=True)
        lse = jnp.log(jnp.sum(jnp.exp(z - m), axis=1, keepdims=True)) + m
        cols.append((z - lse)[:, None, :])
    o_ref[...] = jnp.concatenate(cols, axis=1)


_outk = pl.pallas_call(
    _out_body,
    out_shape=jax.ShapeDtypeStruct((NPK, 8, OP), jnp.float32),
)


def kernel(x, edge_index, W1l, b1, W1r, W2l, b2, W2r):
    edges = edge_index.astype(jnp.int32).reshape(2, NROW, CH).swapaxes(0, 1)
    zeros_h = jnp.zeros((NP, 16), jnp.float32)
    ones_h = jnp.ones((CH, 16), jnp.float32)

    xlp, xrp = _proj(x.reshape(NPK, 8, 128), W1l.T, W1r.T)
    agg1p, degp = _make_seg(True)(xlp.reshape(NN, 16), edges, zeros_h, ones_h)
    hp, dinvp = _hcomb(
        agg1p.reshape(2, NPP, 128), degp.reshape(2, NPP, 128), xrp,
        jnp.tile(b1, 8).reshape(1, 128))
    agg2p = _make_seg(False)(hp.reshape(NN, 16), edges, zeros_h, ones_h)

    w2l_t = jnp.zeros((16, OP), jnp.float32).at[:, :OO].set(W2l.T)
    w2r_t = jnp.zeros((16, OP), jnp.float32).at[:, :OO].set(W2r.T)
    b2p = jnp.full((1, OP), -1e30, jnp.float32).at[0, :OO].set(b2)
    out = _outk(agg2p.reshape(2, NPP, 128), dinvp, hp, w2l_t, w2r_t, b2p)
    return out.reshape(NN, OP)[:, :OO]


# speedup vs baseline: 33.3830x; 1.6574x over previous
import functools

import jax
import jax.numpy as jnp
from jax import lax
from jax.experimental import pallas as pl
from jax.experimental.pallas import tpu as pltpu
from jax.experimental.pallas import tpu_sc as plsc

NN = 10000
NPK = 1250
NP = 10112
NPP = NP // 8
EE = 320000
CH = 128
NROW = EE // CH
NW = 32
BASE = 78
K = 6
NG = BASE // K
RS = NP // 16
OPc = 304
OO = 300


def _seg_body(with_deg, vals, edges, zeros_h, ones_h, *rest):
    if with_deg:
        out_acc, out_deg, src_v, dst_v, rows_v, ones_v, acc, accd, sem_g, sem_sv, sem_sd = rest
    else:
        out_acc, src_v, dst_v, rows_v, ones_v, acc, accd, sem_g, sem_sv, sem_sd = rest
        out_deg = None
    cid = lax.axis_index("c")
    sid = lax.axis_index("s")
    wid = sid * 2 + cid
    pltpu.sync_copy(zeros_h.at[pl.ds(sid * RS, RS)], acc.at[pl.ds(sid * RS, RS)])
    if with_deg:
        pltpu.sync_copy(zeros_h.at[pl.ds(sid * RS, RS)], accd.at[pl.ds(sid * RS, RS)])
        pltpu.sync_copy(ones_h, ones_v)
    pltpu.sync_copy(edges.at[pl.ds(wid * BASE, BASE), 0], src_v.at[pl.ds(0, BASE)])
    pltpu.sync_copy(edges.at[pl.ds(wid * BASE, BASE), 1], dst_v.at[pl.ds(0, BASE)])

    @pl.when(wid < NROW - NW * BASE)
    def _():
        pltpu.sync_copy(edges.at[pl.ds(NW * BASE + wid, 1), 0], src_v.at[pl.ds(BASE, 1)])
        pltpu.sync_copy(edges.at[pl.ds(NW * BASE + wid, 1), 1], dst_v.at[pl.ds(BASE, 1)])

    plsc.subcore_barrier()

    def gather(row, slot):
        pltpu.async_copy(vals.at[src_v.at[row]], rows_v.at[pl.ds(slot * CH, CH)], sem_g)

    def drain_gather():
        pltpu.make_async_copy(
            vals.at[src_v.at[0]], rows_v.at[pl.ds(0, CH)], sem_g).wait()

    def scatter(row, slot):
        pltpu.async_copy(rows_v.at[pl.ds(slot * CH, CH)], acc.at[dst_v.at[row]],
                         sem_sv, add=True)
        if with_deg:
            pltpu.async_copy(ones_v, accd.at[dst_v.at[row]], sem_sd, add=True)

    def drain_scatter():
        pltpu.make_async_copy(
            rows_v.at[pl.ds(0, CH)], acc.at[dst_v.at[0]], sem_sv).wait()
        if with_deg:
            pltpu.make_async_copy(ones_v, accd.at[dst_v.at[0]], sem_sd).wait()

    def group(g, carry):
        s = lax.rem(g, 2)

        @pl.when(g >= 2)
        def _():
            for _k in range(K):
                drain_scatter()

        for k in range(K):
            gather(g * K + k, s * K + k)
        for _k in range(K):
            drain_gather()
        for k in range(K):
            scatter(g * K + k, s * K + k)
        return carry

    lax.fori_loop(0, NG, group, 0)
    for _k in range(2 * K):
        drain_scatter()

    @pl.when(wid < NROW - NW * BASE)
    def _():
        pltpu.async_copy(vals.at[src_v.at[BASE]], rows_v.at[pl.ds(0, CH)], sem_g).wait()
        pltpu.sync_copy(rows_v.at[pl.ds(0, CH)], acc.at[dst_v.at[BASE]], add=True)
        if with_deg:
            pltpu.sync_copy(ones_v, accd.at[dst_v.at[BASE]], add=True)

    plsc.subcore_barrier()
    pltpu.sync_copy(acc.at[pl.ds(sid * RS, RS)], out_acc.at[cid, pl.ds(sid * RS, RS)])
    if with_deg:
        pltpu.sync_copy(accd.at[pl.ds(sid * RS, RS)], out_deg.at[cid, pl.ds(sid * RS, RS)])


@functools.cache
def _make_seg(with_deg):
    mesh = plsc.VectorSubcoreMesh(
        core_axis_name="c", subcore_axis_name="s", num_cores=2, num_subcores=16
    )
    outs = [jax.ShapeDtypeStruct((2, NP, 16), jnp.float32)]
    if with_deg:
        outs.append(jax.ShapeDtypeStruct((2, NP, 16), jnp.float32))
    return pl.kernel(
        functools.partial(_seg_body, with_deg),
        out_type=tuple(outs) if with_deg else outs[0],
        mesh=mesh,
        scratch_types=[
            pltpu.VMEM((BASE + 1, CH), jnp.int32),
            pltpu.VMEM((BASE + 1, CH), jnp.int32),
            pltpu.VMEM((2 * K * CH, 16), jnp.float32),
            pltpu.VMEM((CH, 16), jnp.float32),
            pltpu.VMEM_SHARED((NP, 16), jnp.float32),
            pltpu.VMEM_SHARED((NP, 16), jnp.float32),
            pltpu.SemaphoreType.DMA,
            pltpu.SemaphoreType.DMA,
            pltpu.SemaphoreType.DMA,
        ],
        compiler_params=pltpu.CompilerParams(use_tc_tiling_on_sc=False),
    )


def _perm_body(e_ref, o_ref):
    v = e_ref[...]
    q = v // NPK
    o_ref[...] = (v - q * NPK) * 8 + q


_permk = pl.pallas_call(
    _perm_body,
    out_shape=jax.ShapeDtypeStruct((2, EE), jnp.int32),
)


def _proj_body(x_ref, wl_ref, wr_ref, xl_ref, xr_ref):
    xv = x_ref[...]
    xls, xrs = [], []
    for a in range(8):
        xa = xv[a]
        xls.append(jnp.dot(xa, wl_ref[...], preferred_element_type=jnp.float32))
        xrs.append(jnp.dot(xa, wr_ref[...], preferred_element_type=jnp.float32))
    xl_ref[...] = jnp.concatenate(xls, axis=1)
    xr_ref[...] = jnp.concatenate(xrs, axis=1)


_proj = pl.pallas_call(
    _proj_body,
    out_shape=(
        jax.ShapeDtypeStruct((NPK, 128), jnp.float32),
        jax.ShapeDtypeStruct((NPK, 128), jnp.float32),
    ),
)


def _h_body(a_ref, d_ref, xr_ref, b1_ref, hp_ref, dinv_ref):
    deg = d_ref[0, :NPK] + d_ref[1, :NPK]
    dinv = 1.0 / jnp.maximum(deg, 1.0)
    agg = a_ref[0, :NPK] + a_ref[1, :NPK]
    hp_ref[...] = jnp.maximum(agg * dinv + b1_ref[...] + xr_ref[...], 0.0)
    dinv_ref[...] = dinv


_hcomb = pl.pallas_call(
    _h_body,
    out_shape=(
        jax.ShapeDtypeStruct((NPK, 128), jnp.float32),
        jax.ShapeDtypeStruct((NPK, 128), jnp.float32),
    ),
)


def _out_body(a_ref, dinv_ref, h_ref, w2l_ref, w2r_ref, b2_ref, o_ref):
    m2p = (a_ref[0, :NPK] + a_ref[1, :NPK]) * dinv_ref[...]
    hpv = h_ref[...]
    dn = (((1,), (1,)), ((), ()))
    cols = []
    for a in range(8):
        m2a = m2p[:, 16 * a:16 * (a + 1)]
        ha = hpv[:, 16 * a:16 * (a + 1)]
        z = (lax.dot_general(w2l_ref[...], m2a, dn, preferred_element_type=jnp.float32)
             + lax.dot_general(w2r_ref[...], ha, dn, preferred_element_type=jnp.float32)
             + b2_ref[...])
        m = jnp.max(z, axis=0, keepdims=True)
        lse = jnp.log(jnp.sum(jnp.exp(z - m), axis=0, keepdims=True)) + m
        cols.append(z - lse)
    o_ref[...] = jnp.concatenate(cols, axis=1)


_outk = pl.pallas_call(
    _out_body,
    out_shape=jax.ShapeDtypeStruct((OPc, NN), jnp.float32),
)


def kernel(x, edge_index, W1l, b1, W1r, W2l, b2, W2r):
    pedges = _permk(edge_index.astype(jnp.int32))
    edges = pedges.reshape(2, NROW, CH).swapaxes(0, 1)
    zeros_h = jnp.zeros((NP, 16), jnp.float32)
    ones_h = jnp.ones((CH, 16), jnp.float32)

    xlp, xrp = _proj(x.reshape(8, NPK, 128), W1l.T, W1r.T)
    agg1p, degp = _make_seg(True)(xlp.reshape(NN, 16), edges, zeros_h, ones_h)
    hp, dinvp = _hcomb(
        agg1p.reshape(2, NPP, 128), degp.reshape(2, NPP, 128), xrp,
        jnp.tile(b1, 8).reshape(1, 128))
    agg2p = _make_seg(False)(hp.reshape(NN, 16), edges, zeros_h, ones_h)

    w2l_p = jnp.zeros((OPc, 16), jnp.float32).at[:OO].set(W2l)
    w2r_p = jnp.zeros((OPc, 16), jnp.float32).at[:OO].set(W2r)
    b2c = jnp.full((OPc, 1), -1e30, jnp.float32).at[:OO, 0].set(b2)
    outT = _outk(agg2p.reshape(2, NPP, 128), dinvp, hp, w2l_p, w2r_p, b2c)
    return outT.T[:, :OO]


# speedup vs baseline: 34.9425x; 1.0467x over previous
import functools

import jax
import jax.numpy as jnp
from jax import lax
from jax.experimental import pallas as pl
from jax.experimental.pallas import tpu as pltpu
from jax.experimental.pallas import tpu_sc as plsc

NN = 10000
NPK = 1250
NP = 10112
NPP = NP // 8
EE = 320000
CH = 128
NROW = EE // CH
NW = 32
BASE = 78
GR = 13
NGR = BASE // GR
GB = GR * CH
RS = NP // 16
OPc = 304
OO = 300


def _seg_body(with_deg, vals, edges, zeros_h, ones_h, *rest):
    if with_deg:
        out_acc, out_deg, src_v, dst_v, rows_v, ones_v, acc, accd, sem_g, sem_sv, sem_sd = rest
    else:
        out_acc, src_v, dst_v, rows_v, ones_v, acc, accd, sem_g, sem_sv, sem_sd = rest
        out_deg = None
    cid = lax.axis_index("c")
    sid = lax.axis_index("s")
    wid = sid * 2 + cid
    pltpu.sync_copy(zeros_h.at[pl.ds(sid * RS, RS)], acc.at[pl.ds(sid * RS, RS)])
    if with_deg:
        pltpu.sync_copy(zeros_h.at[pl.ds(sid * RS, RS)], accd.at[pl.ds(sid * RS, RS)])
        pltpu.sync_copy(ones_h, ones_v)
    pltpu.sync_copy(edges.at[0, pl.ds(wid * BASE * CH, BASE * CH)],
                    src_v.at[pl.ds(0, BASE * CH)])
    pltpu.sync_copy(edges.at[1, pl.ds(wid * BASE * CH, BASE * CH)],
                    dst_v.at[pl.ds(0, BASE * CH)])

    @pl.when(wid < NROW - NW * BASE)
    def _():
        pltpu.sync_copy(edges.at[0, pl.ds(NW * BASE * CH + wid * CH, CH)],
                        src_v.at[pl.ds(BASE * CH, CH)])
        pltpu.sync_copy(edges.at[1, pl.ds(NW * BASE * CH + wid * CH, CH)],
                        dst_v.at[pl.ds(BASE * CH, CH)])

    plsc.subcore_barrier()

    def gather(g, buf):
        pltpu.async_copy(vals.at[src_v.at[pl.ds(g * GB, GB)]],
                         rows_v.at[pl.ds(buf * GB, GB)], sem_g)

    def drain_gather():
        pltpu.make_async_copy(
            vals.at[src_v.at[pl.ds(0, GB)]], rows_v.at[pl.ds(0, GB)], sem_g).wait()

    def scatter(g, buf):
        pltpu.async_copy(rows_v.at[pl.ds(buf * GB, GB)],
                         acc.at[dst_v.at[pl.ds(g * GB, GB)]], sem_sv, add=True)
        if with_deg:
            pltpu.async_copy(ones_v, accd.at[dst_v.at[pl.ds(g * GB, GB)]],
                             sem_sd, add=True)

    def drain_scatter():
        pltpu.make_async_copy(
            rows_v.at[pl.ds(0, GB)], acc.at[dst_v.at[pl.ds(0, GB)]], sem_sv).wait()
        if with_deg:
            pltpu.make_async_copy(
                ones_v, accd.at[dst_v.at[pl.ds(0, GB)]], sem_sd).wait()

    gather(0, 0)
    for g in range(NGR):
        buf = g % 2
        drain_gather()
        if g + 1 < NGR:
            if g >= 1:
                drain_scatter()
            gather(g + 1, 1 - buf)
        scatter(g, buf)
    drain_scatter()
    drain_scatter()

    @pl.when(wid < NROW - NW * BASE)
    def _():
        pltpu.async_copy(vals.at[src_v.at[pl.ds(BASE * CH, CH)]],
                         rows_v.at[pl.ds(0, CH)], sem_g).wait()
        pltpu.sync_copy(rows_v.at[pl.ds(0, CH)],
                        acc.at[dst_v.at[pl.ds(BASE * CH, CH)]], add=True)
        if with_deg:
            pltpu.sync_copy(ones_v.at[pl.ds(0, CH)],
                            accd.at[dst_v.at[pl.ds(BASE * CH, CH)]], add=True)

    plsc.subcore_barrier()
    pltpu.sync_copy(acc.at[pl.ds(sid * RS, RS)], out_acc.at[cid, pl.ds(sid * RS, RS)])
    if with_deg:
        pltpu.sync_copy(accd.at[pl.ds(sid * RS, RS)], out_deg.at[cid, pl.ds(sid * RS, RS)])


@functools.cache
def _make_seg(with_deg):
    mesh = plsc.VectorSubcoreMesh(
        core_axis_name="c", subcore_axis_name="s", num_cores=2, num_subcores=16
    )
    outs = [jax.ShapeDtypeStruct((2, NP, 16), jnp.float32)]
    if with_deg:
        outs.append(jax.ShapeDtypeStruct((2, NP, 16), jnp.float32))
    return pl.kernel(
        functools.partial(_seg_body, with_deg),
        out_type=tuple(outs) if with_deg else outs[0],
        mesh=mesh,
        scratch_types=[
            pltpu.VMEM(((BASE + 1) * CH,), jnp.int32),
            pltpu.VMEM(((BASE + 1) * CH,), jnp.int32),
            pltpu.VMEM((2 * GB, 16), jnp.float32),
            pltpu.VMEM((GB, 16), jnp.float32),
            pltpu.VMEM_SHARED((NP, 16), jnp.float32),
            pltpu.VMEM_SHARED((NP, 16), jnp.float32),
            pltpu.SemaphoreType.DMA,
            pltpu.SemaphoreType.DMA,
            pltpu.SemaphoreType.DMA,
        ],
        compiler_params=pltpu.CompilerParams(use_tc_tiling_on_sc=False),
    )


def _perm_body(e_ref, o_ref):
    v = e_ref[...]
    q = (v.astype(jnp.float32) * (1.0 / NPK)).astype(jnp.int32)
    o_ref[...] = (v - q * NPK) * 8 + q


_permk = pl.pallas_call(
    _perm_body,
    out_shape=jax.ShapeDtypeStruct((2, EE), jnp.int32),
)


def _proj_body(x_ref, wl_ref, wr_ref, xl_ref, xr_ref):
    xv = x_ref[...]
    dn = (((1,), (1,)), ((), ()))
    xls, xrs = [], []
    for a in range(8):
        xa = xv[a]
        xls.append(lax.dot_general(xa, wl_ref[...], dn, preferred_element_type=jnp.float32))
        xrs.append(lax.dot_general(xa, wr_ref[...], dn, preferred_element_type=jnp.float32))
    xl_ref[...] = jnp.concatenate(xls, axis=1)
    xr_ref[...] = jnp.concatenate(xrs, axis=1)


_proj = pl.pallas_call(
    _proj_body,
    out_shape=(
        jax.ShapeDtypeStruct((NPK, 128), jnp.float32),
        jax.ShapeDtypeStruct((NPK, 128), jnp.float32),
    ),
)


def _h_body(a_ref, d_ref, xr_ref, b1_ref, hp_ref, dinv_ref):
    deg = d_ref[0, :NPK] + d_ref[1, :NPK]
    dinv = 1.0 / jnp.maximum(deg, 1.0)
    agg = a_ref[0, :NPK] + a_ref[1, :NPK]
    hp_ref[...] = jnp.maximum(agg * dinv + b1_ref[...] + xr_ref[...], 0.0)
    dinv_ref[...] = dinv


_hcomb = pl.pallas_call(
    _h_body,
    out_shape=(
        jax.ShapeDtypeStruct((NPK, 128), jnp.float32),
        jax.ShapeDtypeStruct((NPK, 128), jnp.float32),
    ),
)


def _out_body(a_ref, dinv_ref, h_ref, w2l_ref, w2r_ref, b2_ref, o_ref):
    m2p = (a_ref[0, :NPK] + a_ref[1, :NPK]) * dinv_ref[...]
    hpv = h_ref[...]
    dn = (((1,), (1,)), ((), ()))
    cols = []
    for a in range(8):
        m2a = m2p[:, 16 * a:16 * (a + 1)]
        ha = hpv[:, 16 * a:16 * (a + 1)]
        z = (lax.dot_general(w2l_ref[...], m2a, dn, preferred_element_type=jnp.float32)
             + lax.dot_general(w2r_ref[...], ha, dn, preferred_element_type=jnp.float32)
             + b2_ref[...])
        m = jnp.max(z, axis=0, keepdims=True)
        lse = jnp.log(jnp.sum(jnp.exp(z - m), axis=0, keepdims=True)) + m
        cols.append(z - lse)
    o_ref[...] = jnp.concatenate(cols, axis=1)


_outk = pl.pallas_call(
    _out_body,
    out_shape=jax.ShapeDtypeStruct((OPc, NN), jnp.float32),
)


def kernel(x, edge_index, W1l, b1, W1r, W2l, b2, W2r):
    edges = _permk(edge_index.astype(jnp.int32))
    zeros_h = jnp.zeros((NP, 16), jnp.float32)
    ones_h = jnp.ones((GB, 16), jnp.float32)

    xlp, xrp = _proj(x.reshape(8, NPK, 128), W1l, W1r)
    agg1p, degp = _make_seg(True)(xlp.reshape(NN, 16), edges, zeros_h, ones_h)
    hp, dinvp = _hcomb(
        agg1p.reshape(2, NPP, 128), degp.reshape(2, NPP, 128), xrp,
        jnp.tile(b1, 8).reshape(1, 128))
    agg2p = _make_seg(False)(hp.reshape(NN, 16), edges, zeros_h, ones_h)

    w2l_p = jnp.zeros((OPc, 16), jnp.float32).at[:OO].set(W2l)
    w2r_p = jnp.zeros((OPc, 16), jnp.float32).at[:OO].set(W2r)
    b2c = jnp.full((OPc, 1), -1e30, jnp.float32).at[:OO, 0].set(b2)
    outT = _outk(agg2p.reshape(2, NPP, 128), dinvp, hp, w2l_p, w2r_p, b2c)
    return outT.T[:, :OO]


# speedup vs baseline: 36.1667x; 1.0350x over previous
import functools

import jax
import jax.numpy as jnp
from jax import lax
from jax.experimental import pallas as pl
from jax.experimental.pallas import tpu as pltpu
from jax.experimental.pallas import tpu_sc as plsc

NN = 10000
NPK = 1250
NP = 10112
NPP = NP // 8
EE = 320000
CH = 128
NROW = EE // CH
NW = 32
BASE = 78
GR = 13
NGR = BASE // GR
GB = GR * CH
RS = NP // 16
OPc = 304
OO = 300


def _seg_body(with_deg, vals, edges, zeros_h, ones_h, *rest):
    if with_deg:
        out_acc, out_deg, src_v, dst_v, rows_v, ones_v, acc, accd, sem_g, sem_sv, sem_sd = rest
    else:
        out_acc, src_v, dst_v, rows_v, ones_v, acc, accd, sem_g, sem_sv, sem_sd = rest
        out_deg = None
    cid = lax.axis_index("c")
    sid = lax.axis_index("s")
    wid = sid * 2 + cid
    pltpu.sync_copy(zeros_h.at[pl.ds(sid * RS, RS)], acc.at[pl.ds(sid * RS, RS)])
    if with_deg:
        pltpu.sync_copy(zeros_h.at[pl.ds(sid * RS, RS)], accd.at[pl.ds(sid * RS, RS)])
        pltpu.sync_copy(ones_h, ones_v)
    pltpu.sync_copy(edges.at[0, pl.ds(wid * BASE * CH, BASE * CH)],
                    src_v.at[pl.ds(0, BASE * CH)])
    pltpu.sync_copy(edges.at[1, pl.ds(wid * BASE * CH, BASE * CH)],
                    dst_v.at[pl.ds(0, BASE * CH)])

    @pl.when(wid < NROW - NW * BASE)
    def _():
        pltpu.sync_copy(edges.at[0, pl.ds(NW * BASE * CH + wid * CH, CH)],
                        src_v.at[pl.ds(BASE * CH, CH)])
        pltpu.sync_copy(edges.at[1, pl.ds(NW * BASE * CH + wid * CH, CH)],
                        dst_v.at[pl.ds(BASE * CH, CH)])

    plsc.subcore_barrier()

    def gather(g, buf):
        pltpu.async_copy(vals.at[src_v.at[pl.ds(g * GB, GB)]],
                         rows_v.at[pl.ds(buf * GB, GB)], sem_g)

    def drain_gather():
        pltpu.make_async_copy(
            vals.at[src_v.at[pl.ds(0, GB)]], rows_v.at[pl.ds(0, GB)], sem_g).wait()

    def scatter(g, buf):
        pltpu.async_copy(rows_v.at[pl.ds(buf * GB, GB)],
                         acc.at[dst_v.at[pl.ds(g * GB, GB)]], sem_sv, add=True)
        if with_deg:
            pltpu.async_copy(ones_v, accd.at[dst_v.at[pl.ds(g * GB, GB)]],
                             sem_sd, add=True)

    def drain_scatter():
        pltpu.make_async_copy(
            rows_v.at[pl.ds(0, GB)], acc.at[dst_v.at[pl.ds(0, GB)]], sem_sv).wait()
        if with_deg:
            pltpu.make_async_copy(
                ones_v, accd.at[dst_v.at[pl.ds(0, GB)]], sem_sd).wait()

    gather(0, 0)
    for g in range(NGR):
        buf = g % 2
        drain_gather()
        if g + 1 < NGR:
            if g >= 1:
                drain_scatter()
            gather(g + 1, 1 - buf)
        scatter(g, buf)
    drain_scatter()
    drain_scatter()

    @pl.when(wid < NROW - NW * BASE)
    def _():
        pltpu.async_copy(vals.at[src_v.at[pl.ds(BASE * CH, CH)]],
                         rows_v.at[pl.ds(0, CH)], sem_g).wait()
        pltpu.sync_copy(rows_v.at[pl.ds(0, CH)],
                        acc.at[dst_v.at[pl.ds(BASE * CH, CH)]], add=True)
        if with_deg:
            pltpu.sync_copy(ones_v.at[pl.ds(0, CH)],
                            accd.at[dst_v.at[pl.ds(BASE * CH, CH)]], add=True)

    plsc.subcore_barrier()
    pltpu.sync_copy(acc.at[pl.ds(sid * RS, RS)], out_acc.at[cid, pl.ds(sid * RS, RS)])
    if with_deg:
        pltpu.sync_copy(accd.at[pl.ds(sid * RS, RS)], out_deg.at[cid, pl.ds(sid * RS, RS)])


@functools.cache
def _make_seg(with_deg):
    mesh = plsc.VectorSubcoreMesh(
        core_axis_name="c", subcore_axis_name="s", num_cores=2, num_subcores=16
    )
    outs = [jax.ShapeDtypeStruct((2, NP, 16), jnp.float32)]
    if with_deg:
        outs.append(jax.ShapeDtypeStruct((2, NP, 16), jnp.float32))
    return pl.kernel(
        functools.partial(_seg_body, with_deg),
        out_type=tuple(outs) if with_deg else outs[0],
        mesh=mesh,
        scratch_types=[
            pltpu.VMEM(((BASE + 1) * CH,), jnp.int32),
            pltpu.VMEM(((BASE + 1) * CH,), jnp.int32),
            pltpu.VMEM((2 * GB, 16), jnp.float32),
            pltpu.VMEM((GB, 16), jnp.float32),
            pltpu.VMEM_SHARED((NP, 16), jnp.float32),
            pltpu.VMEM_SHARED((NP, 16), jnp.float32),
            pltpu.SemaphoreType.DMA,
            pltpu.SemaphoreType.DMA,
            pltpu.SemaphoreType.DMA,
        ],
        compiler_params=pltpu.CompilerParams(use_tc_tiling_on_sc=False),
    )


def _permsc_body(edges3, out, buf):
    cid = lax.axis_index("c")
    sid = lax.axis_index("s")
    wid = sid * 2 + cid
    extra = wid < NROW - NW * BASE

    def row(r, carry):
        for k in range(8):
            v = buf[r, pl.ds(16 * k, 16)]
            q = (v.astype(jnp.float32) * (1.0 / NPK)).astype(jnp.int32)
            buf[r, pl.ds(16 * k, 16)] = (v - q * NPK) * 8 + q
        return carry

    for j in range(2):
        pltpu.sync_copy(edges3.at[pl.ds(wid * BASE, BASE), j], buf.at[pl.ds(0, BASE)])

        @pl.when(extra)
        def _():
            pltpu.sync_copy(edges3.at[pl.ds(NW * BASE + wid, 1), j],
                            buf.at[pl.ds(BASE, 1)])

        lax.fori_loop(0, BASE, row, 0)

        @pl.when(extra)
        def _():
            lax.fori_loop(BASE, BASE + 1, row, 0)

        pltpu.sync_copy(buf.at[pl.ds(0, BASE)], out.at[j, pl.ds(wid * BASE, BASE)])

        @pl.when(extra)
        def _():
            pltpu.sync_copy(buf.at[pl.ds(BASE, 1)],
                            out.at[j, pl.ds(NW * BASE + wid, 1)])


@functools.cache
def _make_perm():
    mesh = plsc.VectorSubcoreMesh(
        core_axis_name="c", subcore_axis_name="s", num_cores=2, num_subcores=16
    )
    return pl.kernel(
        _permsc_body,
        out_type=jax.ShapeDtypeStruct((2, NROW, CH), jnp.int32),
        mesh=mesh,
        scratch_types=[pltpu.VMEM((BASE + 1, CH), jnp.int32)],
        compiler_params=pltpu.CompilerParams(use_tc_tiling_on_sc=False),
    )


def _proj_body(x_ref, wl_ref, wr_ref, xl_ref, xr_ref):
    xv = x_ref[...]
    dn = (((1,), (1,)), ((), ()))
    xls, xrs = [], []
    for a in range(8):
        xa = xv[a]
        xls.append(lax.dot_general(xa, wl_ref[...], dn, preferred_element_type=jnp.float32))
        xrs.append(lax.dot_general(xa, wr_ref[...], dn, preferred_element_type=jnp.float32))
    xl_ref[...] = jnp.concatenate(xls, axis=1)
    xr_ref[...] = jnp.concatenate(xrs, axis=1)


_proj = pl.pallas_call(
    _proj_body,
    out_shape=(
        jax.ShapeDtypeStruct((NPK, 128), jnp.float32),
        jax.ShapeDtypeStruct((NPK, 128), jnp.float32),
    ),
)


def _h_body(a_ref, d_ref, xr_ref, b1_ref, hp_ref, dinv_ref):
    deg = d_ref[0, :NPK] + d_ref[1, :NPK]
    dinv = 1.0 / jnp.maximum(deg, 1.0)
    agg = a_ref[0, :NPK] + a_ref[1, :NPK]
    hp_ref[...] = jnp.maximum(agg * dinv + b1_ref[...] + xr_ref[...], 0.0)
    dinv_ref[...] = dinv


_hcomb = pl.pallas_call(
    _h_body,
    out_shape=(
        jax.ShapeDtypeStruct((NPK, 128), jnp.float32),
        jax.ShapeDtypeStruct((NPK, 128), jnp.float32),
    ),
)


def _out_body(a_ref, dinv_ref, h_ref, w2l_ref, w2r_ref, b2_ref, o_ref):
    m2p = (a_ref[0, :NPK] + a_ref[1, :NPK]) * dinv_ref[...]
    hpv = h_ref[...]
    dn = (((1,), (1,)), ((), ()))
    cols = []
    for a in range(8):
        m2a = m2p[:, 16 * a:16 * (a + 1)]
        ha = hpv[:, 16 * a:16 * (a + 1)]
        z = (lax.dot_general(w2l_ref[...], m2a, dn, preferred_element_type=jnp.float32)
             + lax.dot_general(w2r_ref[...], ha, dn, preferred_element_type=jnp.float32)
             + b2_ref[...])
        m = jnp.max(z, axis=0, keepdims=True)
        lse = jnp.log(jnp.sum(jnp.exp(z - m), axis=0, keepdims=True)) + m
        cols.append(z - lse)
    o_ref[...] = jnp.concatenate(cols, axis=1)


_outk = pl.pallas_call(
    _out_body,
    out_shape=jax.ShapeDtypeStruct((OPc, NN), jnp.float32),
)


def kernel(x, edge_index, W1l, b1, W1r, W2l, b2, W2r):
    edges3 = edge_index.astype(jnp.int32).reshape(2, NROW, CH).swapaxes(0, 1)
    edges = _make_perm()(edges3).reshape(2, EE)
    zeros_h = jnp.zeros((NP, 16), jnp.float32)
    ones_h = jnp.ones((GB, 16), jnp.float32)

    xlp, xrp = _proj(x.reshape(8, NPK, 128), W1l, W1r)
    agg1p, degp = _make_seg(True)(xlp.reshape(NN, 16), edges, zeros_h, ones_h)
    hp, dinvp = _hcomb(
        agg1p.reshape(2, NPP, 128), degp.reshape(2, NPP, 128), xrp,
        jnp.tile(b1, 8).reshape(1, 128))
    agg2p = _make_seg(False)(hp.reshape(NN, 16), edges, zeros_h, ones_h)

    w2l_p = jnp.zeros((OPc, 16), jnp.float32).at[:OO].set(W2l)
    w2r_p = jnp.zeros((OPc, 16), jnp.float32).at[:OO].set(W2r)
    b2c = jnp.full((OPc, 1), -1e30, jnp.float32).at[:OO, 0].set(b2)
    outT = _outk(agg2p.reshape(2, NPP, 128), dinvp, hp, w2l_p, w2r_p, b2c)
    return outT.T[:, :OO]
